# baseline (speedup 1.0000x reference)
import jax
import jax.numpy as jnp
from jax import lax
from jax.experimental import pallas as pl
from jax.experimental.pallas import tpu as pltpu
from jax.experimental.pallas import tpu_sc as plsc

V = 2
N = 10000
D = 128
E = 320000
OUT = 128
HID = 2 * OUT

NP = 10240
NT = 16
IR = 128
RPT_IDX = 160
EPT = RPT_IDX * IR
EP = EPT * NT
RPV = EP // IR
NODES_PT = NP // NT

_f32 = jnp.float32


def _sc_mesh():
    return plsc.VectorSubcoreMesh(core_axis_name="c", subcore_axis_name="s")


def _deg_body(dst2d, ones_hbm, out, idxd, ones_v, acc):
    c = lax.axis_index("c")
    s = lax.axis_index("s")
    base = c * RPV + s * RPT_IDX
    pltpu.sync_copy(ones_hbm, ones_v)

    def fill(j, carry):
        pltpu.sync_copy(ones_v, acc.at[pl.ds(s * NODES_PT + j * IR, IR)])
        return carry

    lax.fori_loop(0, NODES_PT // IR, fill, 0)
    plsc.subcore_barrier()

    def group(g, carry):
        pltpu.sync_copy(dst2d.at[pl.ds(base + g * IG_ROWS, IG_ROWS)], idxd)

        def step(r, carry2):
            pltpu.sync_copy(ones_v, acc.at[idxd.at[r]], add=True)
            return carry2

        lax.fori_loop(0, IG_ROWS, step, 0)
        return carry

    lax.fori_loop(0, IGROUPS, group, 0)
    plsc.subcore_barrier()
    pltpu.sync_copy(acc.at[pl.ds(s * NODES_PT, NODES_PT)],
                    out.at[pl.ds(c * NP + s * NODES_PT, NODES_PT)])


def _make_deg_kernel():
    return pl.kernel(
        _deg_body,
        out_type=jax.ShapeDtypeStruct((V * NP, D), _f32),
        mesh=_sc_mesh(),
        scratch_types=[
            pltpu.VMEM((IG_ROWS, IR), jnp.int32),
            pltpu.VMEM((IR, D), _f32),
            pltpu.VMEM_SHARED((NP, D), _f32),
        ],
    )


IG_ROWS = 32
IGROUPS = RPT_IDX // IG_ROWS


IGR = 16


def _spmm_body(tab, src2d, dst2d, out, idxs, idxd, rows, semg, semi, acc):
    c = lax.axis_index("c")
    s = lax.axis_index("s")
    base = c * RPV + s * RPT_IDX
    pltpu.sync_copy(tab.at[pl.ds(c * NP + s * NODES_PT, NODES_PT)],
                    acc.at[pl.ds(s * NODES_PT, NODES_PT)])
    pltpu.sync_copy(src2d.at[pl.ds(base, IGR)], idxs.at[0])
    pltpu.sync_copy(dst2d.at[pl.ds(base, IGR)], idxd.at[0])
    plsc.subcore_barrier()

    pltpu.async_copy(tab.at[idxs.at[0, 0]], rows.at[0], semg.at[0])
    pltpu.async_copy(tab.at[idxs.at[0, 1]], rows.at[1], semg.at[1])

    def step(r, carry):
        b = lax.rem(r, 2)
        g = lax.div(r, IGR)
        slot = lax.rem(g, 2)
        rr = lax.rem(r, IGR)

        @pl.when(jnp.logical_and(rr == 0, r + IGR < RPT_IDX))
        def _():
            nslot = lax.rem(g + 1, 2)
            off = base + (g + 1) * IGR
            pltpu.async_copy(src2d.at[pl.ds(off, IGR)], idxs.at[nslot], semi)
            pltpu.async_copy(dst2d.at[pl.ds(off, IGR)], idxd.at[nslot], semi)

        pltpu.make_async_copy(tab.at[pl.ds(0, IR)], rows.at[b],
                              semg.at[b]).wait()
        pltpu.sync_copy(rows.at[b], acc.at[idxd.at[slot, rr]], add=True)

        @pl.when(r + 2 < RPT_IDX)
        def _():
            r2 = r + 2
            slot2 = lax.rem(lax.div(r2, IGR), 2)
            rr2 = lax.rem(r2, IGR)

            @pl.when(rr == IGR - 2)
            def _():
                pltpu.make_async_copy(src2d.at[pl.ds(base, IGR)],
                                      idxs.at[slot2], semi).wait()
                pltpu.make_async_copy(dst2d.at[pl.ds(base, IGR)],
                                      idxd.at[slot2], semi).wait()

            pltpu.async_copy(tab.at[idxs.at[slot2, rr2]], rows.at[b],
                             semg.at[b])

        return carry

    lax.fori_loop(0, RPT_IDX, step, 0)
    plsc.subcore_barrier()
    pltpu.sync_copy(acc.at[pl.ds(s * NODES_PT, NODES_PT)],
                    out.at[pl.ds(c * NP + s * NODES_PT, NODES_PT)])


def _make_spmm_kernel():
    return pl.kernel(
        _spmm_body,
        out_type=jax.ShapeDtypeStruct((V * NP, D), _f32),
        mesh=_sc_mesh(),
        scratch_types=[
            pltpu.VMEM((2, IGR, IR), jnp.int32),
            pltpu.VMEM((2, IGR, IR), jnp.int32),
            pltpu.VMEM((2, IR, D), _f32),
            pltpu.SemaphoreType.DMA((2,)),
            pltpu.SemaphoreType.DMA,
            pltpu.VMEM_SHARED((NP, D), _f32),
        ],
    )


_BN = 2000
_NB = N // _BN


def _prep_body(degp, x, xs):
    dinv = lax.rsqrt(degp[0][:, 0:1])
    xs[0] = x[0] * dinv


def _prep(deg_part, features):
    return pl.pallas_call(
        _prep_body,
        grid=(V, _NB),
        in_specs=[
            pl.BlockSpec((1, _BN, D), lambda v, i: (v, i, 0)),
            pl.BlockSpec((1, _BN, D), lambda v, i: (v, i, 0)),
        ],
        out_specs=pl.BlockSpec((1, _BN, D), lambda v, i: (v, i, 0)),
        out_shape=jax.ShapeDtypeStruct((V, NP, D), _f32),
    )(deg_part, features)


def _mid_body(t1, degp, w1, b1, w2, zs):
    v = pl.program_id(0)
    dinv = lax.rsqrt(degp[0][:, 0:1])
    y = t1[0] * dinv
    bias = b1[pl.ds(v, 1), :]
    h = jnp.dot(y, w1[0], preferred_element_type=_f32) + bias
    h = jnp.maximum(h, 0.0)
    z = jnp.dot(h, w2[0], preferred_element_type=_f32)
    zs[0] = z * dinv


def _mid(t1, deg_part, W1, b1, W2):
    return pl.pallas_call(
        _mid_body,
        grid=(V, _NB),
        in_specs=[
            pl.BlockSpec((1, _BN, D), lambda v, i: (v, i, 0)),
            pl.BlockSpec((1, _BN, D), lambda v, i: (v, i, 0)),
            pl.BlockSpec((1, D, HID), lambda v, i: (v, 0, 0)),
            pl.BlockSpec((V, HID), lambda v, i: (0, 0)),
            pl.BlockSpec((1, HID, OUT), lambda v, i: (v, 0, 0)),
        ],
        out_specs=pl.BlockSpec((1, _BN, D), lambda v, i: (v, i, 0)),
        out_shape=jax.ShapeDtypeStruct((V, NP, D), _f32),
    )(t1, deg_part, W1, b1, W2)


def _gate_body(x, wg1r, bg1, wg2, bg2, g2):
    gf = jnp.sum(x[...], axis=1) * (1.0 / N)
    hp = (jnp.dot(gf[0:1, :], wg1r[0], preferred_element_type=_f32)
          + jnp.dot(gf[1:2, :], wg1r[1], preferred_element_type=_f32))
    h = jnp.maximum(hp + bg1[...][None, :], 0.0)
    logits = jnp.dot(h, wg2[...], preferred_element_type=_f32) + bg2[...][None, :]
    e = jnp.exp(logits - jnp.max(logits))
    g2[...] = e / jnp.sum(e)


def _gate(features, Wg1r, bg1, Wg2, bg2):
    return pl.pallas_call(
        _gate_body,
        out_shape=jax.ShapeDtypeStruct((1, V), _f32),
    )(features, Wg1r, bg1, Wg2, bg2)


def _comb_body(t2, degp, g2, b2, out):
    o0 = t2[0] * lax.rsqrt(degp[0][:, 0:1]) + b2[0][None, :]
    o1 = t2[1] * lax.rsqrt(degp[1][:, 0:1]) + b2[1][None, :]
    out[...] = g2[0:1, 0:1] * o0 + g2[0:1, 1:2] * o1


def _comb(t2, deg_part, g2, b2):
    return pl.pallas_call(
        _comb_body,
        grid=(_NB,),
        in_specs=[
            pl.BlockSpec((V, _BN, D), lambda i: (0, i, 0)),
            pl.BlockSpec((V, _BN, D), lambda i: (0, i, 0)),
            pl.BlockSpec((1, V), lambda i: (0, 0)),
            pl.BlockSpec((V, OUT), lambda i: (0, 0)),
        ],
        out_specs=pl.BlockSpec((_BN, D), lambda i: (i, 0)),
        out_shape=jax.ShapeDtypeStruct((N, D), _f32),
    )(t2, deg_part, g2, b2)


def kernel(features_list, edge_indices, W1, b1, W2, b2, Wg1, bg1, Wg2, bg2):
    i32 = jnp.int32
    src = edge_indices[:, 0, :]
    dst = edge_indices[:, 1, :]
    voff = jnp.array([[0], [NP]], dtype=i32)
    srcp = jnp.concatenate(
        [src + voff, jnp.full((V, EP - E), N, dtype=i32) + voff], axis=1
    ).reshape(V * RPV, IR)
    dstp = jnp.concatenate(
        [dst, jnp.full((V, EP - E), N, dtype=i32)], axis=1
    ).reshape(V * RPV, IR)

    ones_blk = jnp.ones((IR, D), _f32)

    deg_part = _make_deg_kernel()(dstp, ones_blk)
    deg_part = deg_part.reshape(V, NP, D)

    xs = _prep(deg_part, features_list)

    t1 = _make_spmm_kernel()(xs.reshape(V * NP, D), srcp, dstp)
    t1 = t1.reshape(V, NP, D)

    zs = _mid(t1, deg_part, W1, b1, W2)

    t2 = _make_spmm_kernel()(zs.reshape(V * NP, D), srcp, dstp)
    t2 = t2.reshape(V, NP, D)

    g2 = _gate(features_list, Wg1.reshape(V, D, 128), bg1, Wg2, bg2)
    unified = _comb(t2, deg_part, g2, b2)
    return unified, g2.reshape(V)


# speedup vs baseline: 1.1106x; 1.1106x over previous
import jax
import jax.numpy as jnp
from jax import lax
from jax.experimental import pallas as pl
from jax.experimental.pallas import tpu as pltpu
from jax.experimental.pallas import tpu_sc as plsc

V = 2
N = 10000
D = 128
E = 320000
OUT = 128
HID = 2 * OUT

NP = 10240
NT = 16
IR = 128
RPT_IDX = 160
EPT = RPT_IDX * IR
EP = EPT * NT
RPV = EP // IR
NODES_PT = NP // NT

_f32 = jnp.float32


def _sc_mesh():
    return plsc.VectorSubcoreMesh(core_axis_name="c", subcore_axis_name="s")


def _deg_body(dst2d, onestab, out, idxd, ones_v, acc):
    c = lax.axis_index("c")
    s = lax.axis_index("s")
    base = c * RPV + s * RPT_IDX
    pltpu.sync_copy(onestab.at[pl.ds(s * NODES_PT, NODES_PT)],
                    acc.at[pl.ds(s * NODES_PT, NODES_PT)])
    pltpu.sync_copy(onestab.at[pl.ds(0, IR)], ones_v)
    plsc.subcore_barrier()

    def group(g, carry):
        pltpu.sync_copy(dst2d.at[pl.ds(base + g * IG_ROWS, IG_ROWS)], idxd)

        def step(r, carry2):
            pltpu.sync_copy(ones_v, acc.at[idxd.at[r]], add=True)
            return carry2

        lax.fori_loop(0, IG_ROWS, step, 0)
        return carry

    lax.fori_loop(0, IGROUPS, group, 0)
    plsc.subcore_barrier()
    pltpu.sync_copy(acc.at[pl.ds(s * NODES_PT, NODES_PT)],
                    out.at[pl.ds(c * NP + s * NODES_PT, NODES_PT)])


def _make_deg_kernel():
    return pl.kernel(
        _deg_body,
        out_type=jax.ShapeDtypeStruct((V * NP, D), _f32),
        mesh=_sc_mesh(),
        scratch_types=[
            pltpu.VMEM((IG_ROWS, IR), jnp.int32),
            pltpu.VMEM((IR, D), _f32),
            pltpu.VMEM_SHARED((NP, D), _f32),
        ],
    )


IG_ROWS = 32
IGROUPS = RPT_IDX // IG_ROWS


IGR = 16


def _spmm_body(tab, src2d, dst2d, out, idxs, idxd, rows, semg, semi, acc):
    c = lax.axis_index("c")
    s = lax.axis_index("s")
    base = c * RPV + s * RPT_IDX
    pltpu.sync_copy(tab.at[pl.ds(c * NP + s * NODES_PT, NODES_PT)],
                    acc.at[pl.ds(s * NODES_PT, NODES_PT)])
    pltpu.sync_copy(src2d.at[pl.ds(base, IGR)], idxs.at[0])
    pltpu.sync_copy(dst2d.at[pl.ds(base, IGR)], idxd.at[0])
    plsc.subcore_barrier()

    pltpu.async_copy(tab.at[idxs.at[0, 0]], rows.at[0], semg.at[0])
    pltpu.async_copy(tab.at[idxs.at[0, 1]], rows.at[1], semg.at[1])

    def step(r, carry):
        b = lax.rem(r, 2)
        g = lax.div(r, IGR)
        slot = lax.rem(g, 2)
        rr = lax.rem(r, IGR)

        @pl.when(jnp.logical_and(rr == 0, r + IGR < RPT_IDX))
        def _():
            nslot = lax.rem(g + 1, 2)
            off = base + (g + 1) * IGR
            pltpu.async_copy(src2d.at[pl.ds(off, IGR)], idxs.at[nslot], semi)
            pltpu.async_copy(dst2d.at[pl.ds(off, IGR)], idxd.at[nslot], semi)

        pltpu.make_async_copy(tab.at[pl.ds(0, IR)], rows.at[b],
                              semg.at[b]).wait()
        pltpu.sync_copy(rows.at[b], acc.at[idxd.at[slot, rr]], add=True)

        @pl.when(r + 2 < RPT_IDX)
        def _():
            r2 = r + 2
            slot2 = lax.rem(lax.div(r2, IGR), 2)
            rr2 = lax.rem(r2, IGR)

            @pl.when(rr == IGR - 2)
            def _():
                pltpu.make_async_copy(src2d.at[pl.ds(base, IGR)],
                                      idxs.at[slot2], semi).wait()
                pltpu.make_async_copy(dst2d.at[pl.ds(base, IGR)],
                                      idxd.at[slot2], semi).wait()

            pltpu.async_copy(tab.at[idxs.at[slot2, rr2]], rows.at[b],
                             semg.at[b])

        return carry

    lax.fori_loop(0, RPT_IDX, step, 0)
    plsc.subcore_barrier()
    pltpu.sync_copy(acc.at[pl.ds(s * NODES_PT, NODES_PT)],
                    out.at[pl.ds(c * NP + s * NODES_PT, NODES_PT)])


def _make_spmm_kernel():
    return pl.kernel(
        _spmm_body,
        out_type=jax.ShapeDtypeStruct((V * NP, D), _f32),
        mesh=_sc_mesh(),
        scratch_types=[
            pltpu.VMEM((2, IGR, IR), jnp.int32),
            pltpu.VMEM((2, IGR, IR), jnp.int32),
            pltpu.VMEM((2, IR, D), _f32),
            pltpu.SemaphoreType.DMA((2,)),
            pltpu.SemaphoreType.DMA,
            pltpu.VMEM_SHARED((NP, D), _f32),
        ],
    )


_BN = 2048
_NB = NP // _BN


def _prep_body(degp, x, xs):
    dinv = lax.rsqrt(degp[0][:, 0:1])
    xs[0] = x[0] * dinv


def _prep(deg_part, features_pad):
    return pl.pallas_call(
        _prep_body,
        grid=(V, _NB),
        in_specs=[
            pl.BlockSpec((1, _BN, D), lambda v, i: (v, i, 0)),
            pl.BlockSpec((1, _BN, D), lambda v, i: (v, i, 0)),
        ],
        out_specs=pl.BlockSpec((1, _BN, D), lambda v, i: (v, i, 0)),
        out_shape=jax.ShapeDtypeStruct((V, NP, D), _f32),
    )(deg_part, features_pad)


def _mid_body(t1, degp, w1, b1, w2, zs):
    v = pl.program_id(0)
    dinv = lax.rsqrt(degp[0][:, 0:1])
    y = t1[0] * dinv
    bias = b1[pl.ds(v, 1), :]
    h = jnp.dot(y, w1[0], preferred_element_type=_f32) + bias
    h = jnp.maximum(h, 0.0)
    z = jnp.dot(h, w2[0], preferred_element_type=_f32)
    zs[0] = z * dinv


def _mid(t1, deg_part, W1, b1, W2):
    return pl.pallas_call(
        _mid_body,
        grid=(V, _NB),
        in_specs=[
            pl.BlockSpec((1, _BN, D), lambda v, i: (v, i, 0)),
            pl.BlockSpec((1, _BN, D), lambda v, i: (v, i, 0)),
            pl.BlockSpec((1, D, HID), lambda v, i: (v, 0, 0)),
            pl.BlockSpec((V, HID), lambda v, i: (0, 0)),
            pl.BlockSpec((1, HID, OUT), lambda v, i: (v, 0, 0)),
        ],
        out_specs=pl.BlockSpec((1, _BN, D), lambda v, i: (v, i, 0)),
        out_shape=jax.ShapeDtypeStruct((V, NP, D), _f32),
    )(t1, deg_part, W1, b1, W2)


def _gate_body(x, wg1r, bg1, wg2, bg2, g2):
    gf = jnp.sum(x[...], axis=1) * (1.0 / N)
    hp = (jnp.dot(gf[0:1, :], wg1r[0], preferred_element_type=_f32)
          + jnp.dot(gf[1:2, :], wg1r[1], preferred_element_type=_f32))
    h = jnp.maximum(hp + bg1[...][None, :], 0.0)
    logits = jnp.dot(h, wg2[...], preferred_element_type=_f32) + bg2[...][None, :]
    e = jnp.exp(logits - jnp.max(logits))
    g2[...] = e / jnp.sum(e)


def _gate(features, Wg1r, bg1, Wg2, bg2):
    return pl.pallas_call(
        _gate_body,
        out_shape=jax.ShapeDtypeStruct((1, V), _f32),
    )(features, Wg1r, bg1, Wg2, bg2)


def _comb_body(t2, degp, g2, b2, out):
    o0 = t2[0] * lax.rsqrt(degp[0][:, 0:1]) + b2[0][None, :]
    o1 = t2[1] * lax.rsqrt(degp[1][:, 0:1]) + b2[1][None, :]
    out[...] = g2[0:1, 0:1] * o0 + g2[0:1, 1:2] * o1


def _comb(t2, deg_part, g2, b2):
    return pl.pallas_call(
        _comb_body,
        grid=(_NB,),
        in_specs=[
            pl.BlockSpec((V, _BN, D), lambda i: (0, i, 0)),
            pl.BlockSpec((V, _BN, D), lambda i: (0, i, 0)),
            pl.BlockSpec((1, V), lambda i: (0, 0)),
            pl.BlockSpec((V, OUT), lambda i: (0, 0)),
        ],
        out_specs=pl.BlockSpec((_BN, D), lambda i: (i, 0)),
        out_shape=jax.ShapeDtypeStruct((NP, D), _f32),
    )(t2, deg_part, g2, b2)


def kernel(features_list, edge_indices, W1, b1, W2, b2, Wg1, bg1, Wg2, bg2):
    i32 = jnp.int32
    src = edge_indices[:, 0, :]
    dst = edge_indices[:, 1, :]
    voff = jnp.array([[0], [NP]], dtype=i32)
    srcp = jnp.concatenate(
        [src + voff, jnp.full((V, EP - E), N, dtype=i32) + voff], axis=1
    ).reshape(V * RPV, IR)
    dstp = jnp.concatenate(
        [dst, jnp.full((V, EP - E), N, dtype=i32)], axis=1
    ).reshape(V * RPV, IR)

    onestab = jnp.concatenate(
        [jnp.ones((N, D), _f32), jnp.zeros((NP - N, D), _f32)], axis=0)
    features_pad = jnp.pad(features_list, ((0, 0), (0, NP - N), (0, 0)))

    deg_part = _make_deg_kernel()(dstp, onestab)
    deg_part = deg_part.reshape(V, NP, D)

    xs = _prep(deg_part, features_pad)

    t1 = _make_spmm_kernel()(xs.reshape(V * NP, D), srcp, dstp)
    t1 = t1.reshape(V, NP, D)

    zs = _mid(t1, deg_part, W1, b1, W2)

    t2 = _make_spmm_kernel()(zs.reshape(V * NP, D), srcp, dstp)
    t2 = t2.reshape(V, NP, D)

    g2 = _gate(features_list, Wg1.reshape(V, D, 128), bg1, Wg2, bg2)
    unified = _comb(t2, deg_part, g2, b2)
    return unified[:N], g2.reshape(V)
